# Initial kernel scaffold; baseline (speedup 1.0000x reference)
import jax
import jax.numpy as jnp
from jax.experimental import pallas as pl


def kernel(points, features, Wq, bq, Wk, bk, Wv, bv, pe_W1, pe_g1, pe_b1, pe_W2, pe_bias2, at_g1, at_be1, at_W1, at_g2, at_be2, at_W2, at_b2):
    raise NotImplementedError("write your pallas kernel here")


# speedup vs baseline: 18.5173x; 18.5173x over previous
import functools

import jax
import jax.numpy as jnp
import numpy as np
from jax import lax
from jax.experimental import pallas as pl
from jax.experimental.pallas import tpu as pltpu
from jax.experimental.pallas import tpu_sc as plsc

_B, _N, _K = 4, 4096, 16
_C = 32
_QA = 256
_QC = 512
_TBL = 128
_ROWS = _B * _N * _K
_CHUNK = 128


def _phase_a(pts8, feats, Wq, bq, Wk, bk, Wv, bv):
    grid = (_B, _N // _QA)

    def body(pall_ref, pq_ref, f_ref, wq_ref, bq_ref, wk_ref, bk_ref,
             wv_ref, bv_ref, tbl_ref, idx_ref):
        b = pl.program_id(0)
        pall = pall_ref[0]
        pq = pq_ref[0]
        sq_all = jnp.sum(pall * pall, axis=0, keepdims=True)
        sq_q = jnp.sum(pq * pq, axis=0)[:, None]
        dn = (((0,), (0,)), ((), ()))
        cross = lax.dot_general(pq, pall, dn,
                                preferred_element_type=jnp.float32)
        d = (sq_q + sq_all) - 2.0 * cross
        s = -d
        iota = lax.broadcasted_iota(jnp.int32, (_QA, _N), 1)
        big = jnp.int32(2147483647)
        cols = []
        for _ in range(_K):
            m = jnp.max(s, axis=1, keepdims=True)
            ik = jnp.min(jnp.where(s == m, iota, big), axis=1, keepdims=True)
            cols.append(ik)
            s = jnp.where(iota == ik, -jnp.inf, s)
        idx_ref[0] = jnp.concatenate(cols, axis=1) + b * _N

        f = f_ref[0]
        q = lax.dot_general(f, wq_ref[...], dn,
                            preferred_element_type=jnp.float32) + bq_ref[...]
        kf = lax.dot_general(f, wk_ref[...], dn,
                             preferred_element_type=jnp.float32) + bk_ref[...]
        v = lax.dot_general(f, wv_ref[...], dn,
                            preferred_element_type=jnp.float32) + bv_ref[...]
        eye = jnp.eye(8, 3, dtype=jnp.float32)
        xyz = lax.dot_general(pq, eye, dn,
                              preferred_element_type=jnp.float32)
        pad = jnp.zeros((_QA, _TBL - 99), jnp.float32)
        tbl_ref[0] = jnp.concatenate([kf, v, q, xyz, pad], axis=1)

    return pl.pallas_call(
        body,
        grid=grid,
        in_specs=[
            pl.BlockSpec((1, 8, _N), lambda b, i: (b, 0, 0)),
            pl.BlockSpec((1, 8, _QA), lambda b, i: (b, 0, i)),
            pl.BlockSpec((1, _C, _QA), lambda b, i: (b, 0, i)),
            pl.BlockSpec((_C, _C), lambda b, i: (0, 0)),
            pl.BlockSpec((1, _C), lambda b, i: (0, 0)),
            pl.BlockSpec((_C, _C), lambda b, i: (0, 0)),
            pl.BlockSpec((1, _C), lambda b, i: (0, 0)),
            pl.BlockSpec((_C, _C), lambda b, i: (0, 0)),
            pl.BlockSpec((1, _C), lambda b, i: (0, 0)),
        ],
        out_specs=(
            pl.BlockSpec((1, _QA, _TBL), lambda b, i: (b, i, 0)),
            pl.BlockSpec((1, _QA, _K), lambda b, i: (b, i, 0)),
        ),
        out_shape=(
            jax.ShapeDtypeStruct((_B, _N, _TBL), jnp.float32),
            jax.ShapeDtypeStruct((_B, _N, _K), jnp.int32),
        ),
    )(pts8, pts8, feats, Wq, bq, Wk, bk, Wv, bv)


def _sc_gather(tbl, idxf):
    info = plsc.get_sparse_core_info()
    nw = info.num_cores * info.num_subcores
    r_per_w = _ROWS // nw
    nch = r_per_w // _CHUNK
    mesh = plsc.VectorSubcoreMesh(core_axis_name="c", subcore_axis_name="s")

    @functools.partial(
        pl.kernel,
        mesh=mesh,
        out_type=jax.ShapeDtypeStruct((_ROWS, _TBL), jnp.float32),
        scratch_types=[
            pltpu.VMEM((r_per_w,), jnp.int32),
            pltpu.VMEM((_CHUNK, _TBL), jnp.float32),
            pltpu.SemaphoreType.DMA,
        ],
    )
    def gk(tbl_hbm, idx_hbm, out_hbm, idx_v, rows_v, sem):
        wid = lax.axis_index("s") * info.num_cores + lax.axis_index("c")
        base = wid * r_per_w
        pltpu.sync_copy(idx_hbm.at[pl.ds(base, r_per_w)], idx_v)

        def chunk(i, carry):
            off = pl.multiple_of(i * _CHUNK, _CHUNK)
            pltpu.async_copy(tbl_hbm.at[idx_v.at[pl.ds(off, _CHUNK)]],
                             rows_v, sem).wait()
            pltpu.sync_copy(rows_v, out_hbm.at[pl.ds(base + off, _CHUNK)])
            return carry

        lax.fori_loop(0, nch, chunk, 0)

    return gk(tbl, idxf)


def _phase_c(g4, table, pe_W1p, pe_g1p, pe_b1p, pe_W2p, pe_b2,
             at_g1, at_be1, at_W1, at_g2, at_be2, at_W2, at_b2):
    grid = (_B, _N // _QC)
    inv = float(1.0 / np.sqrt(np.float32(1.0 + 1e-5), dtype=np.float32))

    def body(g_ref, t_ref, w1_ref, g1_ref, b1_ref, w2_ref, pb2_ref,
             ag1_ref, ab1_ref, aw1_ref, ag2_ref, ab2_ref, aw2_ref, abb2_ref,
             y_ref):
        g = g_ref[0]
        t = t_ref[0]
        gk = g[:, :, 0:32]
        gv = g[:, :, 32:64]
        gx = g[:, :, 96:104]
        q = t[:, 64:96]
        cx = t[:, 96:104]
        rel = (gx - cx[None]).reshape(_K * _QC, 8)
        r = jnp.dot(rel, w1_ref[...], preferred_element_type=jnp.float32)
        r = jnp.maximum((r * inv) * g1_ref[...] + b1_ref[...], 0.0)
        n_r = jnp.dot(r, w2_ref[...],
                      preferred_element_type=jnp.float32) + pb2_ref[...]
        n_r3 = n_r.reshape(_K, _QC, _C)
        a = (q[None] - gk) + n_r3
        a = jnp.maximum((a * inv) * ag1_ref[...] + ab1_ref[...], 0.0)
        a = a.reshape(_K * _QC, _C)
        a = jnp.dot(a, aw1_ref[...], preferred_element_type=jnp.float32)
        a = jnp.maximum((a * inv) * ag2_ref[...] + ab2_ref[...], 0.0)
        a = jnp.dot(a, aw2_ref[...],
                    preferred_element_type=jnp.float32) + abb2_ref[...]
        a = a.reshape(_K, _QC, _C)
        m = jnp.max(a, axis=0, keepdims=True)
        e = jnp.exp(a - m)
        attn = e / jnp.sum(e, axis=0, keepdims=True)
        y_ref[0] = jnp.sum((gv + n_r3) * attn, axis=0)

    wspec = lambda shape: pl.BlockSpec(shape, lambda b, i: tuple(0 for _ in shape))
    return pl.pallas_call(
        body,
        grid=grid,
        in_specs=[
            pl.BlockSpec((1, _K, _QC, _TBL), lambda b, i: (b, 0, i, 0)),
            pl.BlockSpec((1, _QC, _TBL), lambda b, i: (b, i, 0)),
            wspec((8, 8)),
            wspec((1, 8)),
            wspec((1, 8)),
            wspec((8, _C)),
            wspec((1, _C)),
            wspec((1, _C)),
            wspec((1, _C)),
            wspec((_C, _C)),
            wspec((1, _C)),
            wspec((1, _C)),
            wspec((_C, _C)),
            wspec((1, _C)),
        ],
        out_specs=pl.BlockSpec((1, _QC, _C), lambda b, i: (b, i, 0)),
        out_shape=jax.ShapeDtypeStruct((_B, _N, _C), jnp.float32),
    )(g4, table, pe_W1p, pe_g1p, pe_b1p, pe_W2p, pe_b2,
      at_g1, at_be1, at_W1, at_g2, at_be2, at_W2, at_b2)


def kernel(points, features, Wq, bq, Wk, bk, Wv, bv, pe_W1, pe_g1, pe_b1,
           pe_W2, pe_bias2, at_g1, at_be1, at_W1, at_g2, at_be2, at_W2,
           at_b2):
    pts_t = jnp.transpose(points, (0, 2, 1))
    pts8 = jnp.concatenate(
        [pts_t, jnp.zeros((_B, 5, _N), jnp.float32)], axis=1)
    table, idx = _phase_a(pts8, features,
                          Wq, bq.reshape(1, _C), Wk, bk.reshape(1, _C),
                          Wv, bv.reshape(1, _C))
    idx_flat = jnp.transpose(idx, (0, 2, 1)).reshape(-1)
    gathered = _sc_gather(table.reshape(_B * _N, _TBL), idx_flat)
    g4 = gathered.reshape(_B, _K, _N, _TBL)

    z = jnp.zeros((8, 8), jnp.float32)
    pe_W1p = z.at[:3, :3].set(pe_W1)
    pe_g1p = jnp.zeros((1, 8), jnp.float32).at[0, :3].set(pe_g1)
    pe_b1p = jnp.zeros((1, 8), jnp.float32).at[0, :3].set(pe_b1)
    pe_W2p = jnp.zeros((8, _C), jnp.float32).at[:3, :].set(pe_W2)

    y = _phase_c(g4, table, pe_W1p, pe_g1p, pe_b1p, pe_W2p,
                 pe_bias2.reshape(1, _C), at_g1.reshape(1, _C),
                 at_be1.reshape(1, _C), at_W1, at_g2.reshape(1, _C),
                 at_be2.reshape(1, _C), at_W2, at_b2.reshape(1, _C))
    return (points, jnp.transpose(y, (0, 2, 1)))
